# Initial kernel scaffold; baseline (speedup 1.0000x reference)
import jax
import jax.numpy as jnp
from jax.experimental import pallas as pl


def kernel(h, x, edge_attr, edge_index, mask_ligand, params):
    raise NotImplementedError("write your pallas kernel here")


# speedup vs baseline: 7.6762x; 7.6762x over previous
import functools
import numpy as np
import jax
import jax.numpy as jnp
from jax import lax
from jax.experimental import pallas as pl
from jax.experimental.pallas import tpu as pltpu

N = 10000
E = 160000
HID = 128
HEADS = 16
DH = 8
NRG = 20
EFD = 4
RFD = NRG * 4

BE = 2000
BN = 2000

_OFF = np.linspace(0.0, 10.0, NRG).astype(np.float32)
_COEFF = np.float32(-0.5 / (_OFF[1] - _OFF[0]) ** 2)
_A4 = np.zeros((EFD, RFD), np.float32)
_G20 = np.zeros((NRG, RFD), np.float32)
for _a in range(EFD):
    for _g in range(NRG):
        _A4[_a, _a * NRG + _g] = 1.0
        _G20[_g, _a * NRG + _g] = 1.0
_HS = np.zeros((HID, HEADS), np.float32)
_EXH = np.zeros((HEADS, HID), np.float32)
for _h in range(HEADS):
    for _j in range(DH):
        _HS[_h * DH + _j, _h] = 1.0
        _EXH[_h, _h * DH + _j] = 1.0
_P48 = np.zeros((HEADS, 48), np.float32)
_Q48 = np.zeros((16, 48), np.float32)
_M48 = np.zeros((48, 8), np.float32)
for _h in range(HEADS):
    for _c in range(3):
        _P48[_h, 3 * _h + _c] = 1.0
        _Q48[_c, 3 * _h + _c] = 1.0
        _M48[3 * _h + _c, _c] = 1.0 / HEADS
_ISQ = np.float32(1.0 / np.sqrt(DH))


def _ln(v, g, b):
    mu = jnp.mean(v, -1, keepdims=True)
    var = jnp.mean((v - mu) ** 2, -1, keepdims=True)
    return (v - mu) / jnp.sqrt(var + 1e-5) * g + b


def _dot(a, b):
    return jnp.dot(a, b, preferred_element_type=jnp.float32)


def _mlp128_body(h_ref, w1, b1, g1, be1, w2, b2, o_ref):
    v = _dot(h_ref[...], w1[...]) + b1[...]
    v = jnp.maximum(_ln(v, g1[...], be1[...]), 0.0)
    o_ref[...] = _dot(v, w2[...]) + b2[...]


def _mlp128(hin, p):
    return pl.pallas_call(
        _mlp128_body,
        grid=(N // BN,),
        in_specs=[
            pl.BlockSpec((BN, HID), lambda i: (i, 0)),
            pl.BlockSpec((HID, HID), lambda i: (0, 0)),
            pl.BlockSpec((1, HID), lambda i: (0, 0)),
            pl.BlockSpec((1, HID), lambda i: (0, 0)),
            pl.BlockSpec((1, HID), lambda i: (0, 0)),
            pl.BlockSpec((HID, HID), lambda i: (0, 0)),
            pl.BlockSpec((1, HID), lambda i: (0, 0)),
        ],
        out_specs=pl.BlockSpec((BN, HID), lambda i: (i, 0)),
        out_shape=jax.ShapeDtypeStruct((N, HID), jnp.float32),
    )(hin, p['W1'], p['b1'][None, :], p['g'][None, :], p['be'][None, :],
      p['W2'], p['b2'][None, :])


def _rfeat(ea, rel, off, a4, g20):
    d2 = jnp.sum(rel * rel, -1, keepdims=True)
    dist = jnp.sqrt(d2)
    sm = jnp.exp(_COEFF * (dist - off) ** 2)
    return _dot(ea, a4) * _dot(sm, g20)


def _edge1_body(ea_ref, dd_ref, ss_ref,
                off, a4, g20, hs_m, exh,
                w1e, w1r, w1d, w1s, b1,
                gk, bek, w2k, b2k, gv, bev, w2v, b2v,
                eww, ewb,
                ex_ref, wv_ref, rel_ref):
    ea = ea_ref[...]
    hd = dd_ref[:, 0:HID]
    qd = dd_ref[:, HID:2 * HID]
    xd = dd_ref[:, 2 * HID:2 * HID + 16]
    hs = ss_ref[:, 0:HID]
    xs = ss_ref[:, HID:HID + 16]
    rel = xd - xs
    rf = _rfeat(ea, rel, off[...], a4[...], g20[...])
    pre = (_dot(ea, w1e[...]) + _dot(rf, w1r[...])
           + _dot(hd, w1d[...]) + _dot(hs, w1s[...])) + b1[...]
    pk = pre[:, :HID]
    pv = pre[:, HID:]
    k = _dot(jnp.maximum(_ln(pk, gk[...], bek[...]), 0.0), w2k[...]) + b2k[...]
    v = _dot(jnp.maximum(_ln(pv, gv[...], bev[...]), 0.0), w2v[...]) + b2v[...]
    ewl = jnp.sum(rf * eww[...], -1, keepdims=True) + ewb[...]
    v = v * (1.0 / (1.0 + jnp.exp(-ewl)))
    ex = jnp.exp(_dot(qd * k, hs_m[...]) * _ISQ)
    ex_ref[...] = ex
    wv_ref[...] = _dot(ex, exh[...]) * v
    rel_ref[...] = rel


def _edge2_body(ea_ref, rel_ref, dd_ref, ss_ref,
                off, a4, g20, hs_m, p48, q48,
                w1e, w1r, w1d, w1s, b1,
                gk, bek, w2k, b2k, gv, bev, w2v, b2v,
                eww, ewb,
                ex_ref, wv_ref):
    ea = ea_ref[...]
    rel = rel_ref[...]
    hd = dd_ref[:, 0:HID]
    qd = dd_ref[:, HID:2 * HID]
    hs = ss_ref[...]
    rf = _rfeat(ea, rel, off[...], a4[...], g20[...])
    pre = (_dot(ea, w1e[...]) + _dot(rf, w1r[...])
           + _dot(hd, w1d[...]) + _dot(hs, w1s[...])) + b1[...]
    pk = pre[:, :HID]
    pv = pre[:, HID:]
    k = _dot(jnp.maximum(_ln(pk, gk[...], bek[...]), 0.0), w2k[...]) + b2k[...]
    v = _dot(jnp.maximum(_ln(pv, gv[...], bev[...]), 0.0), w2v[...]) + b2v[...]
    ewl = jnp.sum(rf * eww[...], -1, keepdims=True) + ewb[...]
    v = v * (1.0 / (1.0 + jnp.exp(-ewl)))
    ex = jnp.exp(_dot(qd * k, hs_m[...]) * _ISQ)
    ex_ref[...] = ex
    wv_ref[...] = _dot(ex * v, p48[...]) * _dot(rel, q48[...])


def _node1_body(exs_ref, wvs_ref, h_ref, exh,
                w1a, w1b, b1, g1, be1, w2, b2, ho_ref):
    exb = _dot(exs_ref[...], exh[...])
    out1 = wvs_ref[...] / (exb + 1e-16)
    pre = _dot(out1, w1a[...]) + _dot(h_ref[...], w1b[...]) + b1[...]
    o = _dot(jnp.maximum(_ln(pre, g1[...], be1[...]), 0.0), w2[...]) + b2[...]
    ho_ref[...] = o + h_ref[...]


def _node2_body(exs_ref, wvs_ref, x_ref, m_ref, p48, m48, xo_ref):
    den = _dot(exs_ref[...], p48[...])
    o = wvs_ref[...] / (den + 1e-16)
    delta = _dot(o, m48[...])
    xo_ref[...] = x_ref[:, :8] + delta * m_ref[...]


_FULL = lambda r, c: pl.BlockSpec((r, c), lambda i: (0, 0))
_ROWB = lambda b, c: pl.BlockSpec((b, c), lambda i: (i, 0))


def _edge1_call(ea, dd, ss, consts, w):
    return pl.pallas_call(
        _edge1_body,
        grid=(E // BE,),
        in_specs=[
            _ROWB(BE, EFD), _ROWB(BE, 2 * HID + 16), _ROWB(BE, HID + 16),
            _FULL(1, NRG), _FULL(EFD, RFD), _FULL(NRG, RFD),
            _FULL(HID, HEADS), _FULL(HEADS, HID),
            _FULL(EFD, 2 * HID), _FULL(RFD, 2 * HID),
            _FULL(HID, 2 * HID), _FULL(HID, 2 * HID), _FULL(1, 2 * HID),
            _FULL(1, HID), _FULL(1, HID), _FULL(HID, HID), _FULL(1, HID),
            _FULL(1, HID), _FULL(1, HID), _FULL(HID, HID), _FULL(1, HID),
            _FULL(1, RFD), _FULL(1, 1),
        ],
        out_specs=[_ROWB(BE, HEADS), _ROWB(BE, HID), _ROWB(BE, 16)],
        out_shape=[
            jax.ShapeDtypeStruct((E, HEADS), jnp.float32),
            jax.ShapeDtypeStruct((E, HID), jnp.float32),
            jax.ShapeDtypeStruct((E, 16), jnp.float32),
        ],
    )(ea, dd, ss, *consts, *w)


def _edge2_call(ea, rel, dd, ss, consts, w):
    return pl.pallas_call(
        _edge2_body,
        grid=(E // BE,),
        in_specs=[
            _ROWB(BE, EFD), _ROWB(BE, 16), _ROWB(BE, 2 * HID), _ROWB(BE, HID),
            _FULL(1, NRG), _FULL(EFD, RFD), _FULL(NRG, RFD),
            _FULL(HID, HEADS), _FULL(HEADS, 48), _FULL(16, 48),
            _FULL(EFD, 2 * HID), _FULL(RFD, 2 * HID),
            _FULL(HID, 2 * HID), _FULL(HID, 2 * HID), _FULL(1, 2 * HID),
            _FULL(1, HID), _FULL(1, HID), _FULL(HID, HID), _FULL(1, HID),
            _FULL(1, HID), _FULL(1, HID), _FULL(HID, HEADS), _FULL(1, HEADS),
            _FULL(1, RFD), _FULL(1, 1),
        ],
        out_specs=[_ROWB(BE, HEADS), _ROWB(BE, 48)],
        out_shape=[
            jax.ShapeDtypeStruct((E, HEADS), jnp.float32),
            jax.ShapeDtypeStruct((E, 48), jnp.float32),
        ],
    )(ea, rel, dd, ss, *consts, *w)


def _split_w1(p, dout):
    w1 = p['W1']
    return (w1[0:EFD], w1[EFD:EFD + RFD],
            w1[EFD + RFD:EFD + RFD + HID], w1[EFD + RFD + HID:])


def _edge_weights(pk, pv, eww, ewb):
    ke, kr, kd, ks = _split_w1(pk, HID)
    ve, vr, vd, vs = _split_w1(pv, None)
    w1e = jnp.concatenate([ke, ve], 1)
    w1r = jnp.concatenate([kr, vr], 1)
    w1d = jnp.concatenate([kd, vd], 1)
    w1s = jnp.concatenate([ks, vs], 1)
    b1 = jnp.concatenate([pk['b1'], pv['b1']])[None, :]
    return [w1e, w1r, w1d, w1s, b1,
            pk['g'][None, :], pk['be'][None, :], pk['W2'], pk['b2'][None, :],
            pv['g'][None, :], pv['be'][None, :], pv['W2'], pv['b2'][None, :],
            eww.reshape(1, RFD), ewb.reshape(1, 1)]


def kernel(h, x, edge_attr, edge_index, mask_ligand, params):
    p = params
    src = edge_index[0]
    dst = edge_index[1]
    xpad = jnp.concatenate([x, jnp.zeros((N, 13), jnp.float32)], 1)

    off = jnp.asarray(_OFF)[None, :]
    a4 = jnp.asarray(_A4)
    g20 = jnp.asarray(_G20)
    hs_m = jnp.asarray(_HS)
    exh = jnp.asarray(_EXH)
    p48 = jnp.asarray(_P48)
    q48 = jnp.asarray(_Q48)
    m48 = jnp.asarray(_M48)

    q1 = _mlp128(h, p['x2h_hq'])
    dd1 = jnp.concatenate([jnp.take(h, dst, 0), jnp.take(q1, dst, 0),
                           jnp.take(xpad, dst, 0)], 1)
    ss1 = jnp.concatenate([jnp.take(h, src, 0), jnp.take(xpad, src, 0)], 1)
    w1 = _edge_weights(p['x2h_hk'], p['x2h_hv'], p['x2h_ew_W'], p['x2h_ew_b'])
    ex1, wv1, rel = _edge1_call(edge_attr, dd1, ss1,
                                (off, a4, g20, hs_m, exh), w1)
    exs1 = jax.ops.segment_sum(ex1, dst, num_segments=N)
    wvs1 = jax.ops.segment_sum(wv1, dst, num_segments=N)

    po = p['x2h_out']
    hout = pl.pallas_call(
        _node1_body,
        grid=(N // BN,),
        in_specs=[
            _ROWB(BN, HEADS), _ROWB(BN, HID), _ROWB(BN, HID),
            _FULL(HEADS, HID),
            _FULL(HID, HID), _FULL(HID, HID), _FULL(1, HID),
            _FULL(1, HID), _FULL(1, HID), _FULL(HID, HID), _FULL(1, HID),
        ],
        out_specs=_ROWB(BN, HID),
        out_shape=jax.ShapeDtypeStruct((N, HID), jnp.float32),
    )(exs1, wvs1, h, exh, po['W1'][:HID], po['W1'][HID:], po['b1'][None, :],
      po['g'][None, :], po['be'][None, :], po['W2'], po['b2'][None, :])

    q2 = _mlp128(hout, p['h2x_xq'])
    dd2 = jnp.concatenate([jnp.take(hout, dst, 0), jnp.take(q2, dst, 0)], 1)
    ss2 = jnp.take(hout, src, 0)
    w2 = _edge_weights(p['h2x_xk'], p['h2x_xv'], p['h2x_ew_W'], p['h2x_ew_b'])
    ex2, wv2 = _edge2_call(edge_attr, rel, dd2, ss2,
                           (off, a4, g20, hs_m, p48, q48), w2)
    exs2 = jax.ops.segment_sum(ex2, dst, num_segments=N)
    wvs2 = jax.ops.segment_sum(wv2, dst, num_segments=N)

    x8 = pl.pallas_call(
        _node2_body,
        grid=(N // BN,),
        in_specs=[
            _ROWB(BN, HEADS), _ROWB(BN, 48), _ROWB(BN, 16), _ROWB(BN, 8),
            _FULL(HEADS, 48), _FULL(48, 8),
        ],
        out_specs=_ROWB(BN, 8),
        out_shape=jax.ShapeDtypeStruct((N, 8), jnp.float32),
    )(exs2, wvs2, xpad,
      jnp.broadcast_to(mask_ligand[:, None], (N, 8)), p48, m48)

    return (hout, x8[:, :3])


# speedup vs baseline: 19.5143x; 2.5422x over previous
import functools
import numpy as np
import jax
import jax.numpy as jnp
from jax import lax
from jax.experimental import pallas as pl
from jax.experimental.pallas import tpu as pltpu
from jax.experimental.pallas import tpu_sc as plsc

N = 10000
E = 160000
HID = 128
HEADS = 16
DH = 8
NRG = 20
EFD = 4
RFD = NRG * 4

BE = 2000
BN = 2000

_OFF = np.linspace(0.0, 10.0, NRG).astype(np.float32)
_COEFF = np.float32(-0.5 / (_OFF[1] - _OFF[0]) ** 2)
_A4 = np.zeros((EFD, RFD), np.float32)
_G20 = np.zeros((NRG, RFD), np.float32)
for _a in range(EFD):
    for _g in range(NRG):
        _A4[_a, _a * NRG + _g] = 1.0
        _G20[_g, _a * NRG + _g] = 1.0
_HS = np.zeros((HID, HEADS), np.float32)
_EXH = np.zeros((HEADS, HID), np.float32)
for _h in range(HEADS):
    for _j in range(DH):
        _HS[_h * DH + _j, _h] = 1.0
        _EXH[_h, _h * DH + _j] = 1.0
_P48 = np.zeros((HEADS, 48), np.float32)
_Q48 = np.zeros((16, 48), np.float32)
_M48 = np.zeros((48, 8), np.float32)
for _h in range(HEADS):
    for _c in range(3):
        _P48[_h, 3 * _h + _c] = 1.0
        _Q48[_c, 3 * _h + _c] = 1.0
        _M48[3 * _h + _c, _c] = 1.0 / HEADS
_ISQ = np.float32(1.0 / np.sqrt(DH))


_CHW = 128
_NCH = E // _CHW
_NW = 32

_SC_MESH = dict(core_axis_name="c", subcore_axis_name="s")


def _sc_gather(table, idxm, d):
    @functools.partial(
        pl.kernel,
        out_type=jax.ShapeDtypeStruct((E, d), jnp.float32),
        mesh=plsc.VectorSubcoreMesh(**_SC_MESH),
        compiler_params=pltpu.CompilerParams(use_tc_tiling_on_sc=False),
        scratch_types=[
            pltpu.VMEM((_CHW,), jnp.int32),
            pltpu.VMEM((_CHW, d), jnp.float32),
            pltpu.SemaphoreType.DMA,
        ],
    )
    def k(table_hbm, idxm_hbm, out_hbm, idx_v, rows_v, sem):
        wid = lax.axis_index("s") * 2 + lax.axis_index("c")
        nch = (_NCH - wid + (_NW - 1)) // _NW

        def body(t, carry):
            ch = wid + t * _NW
            pltpu.sync_copy(idxm_hbm.at[ch], idx_v)
            pltpu.async_copy(table_hbm.at[idx_v], rows_v, sem).wait()
            pltpu.sync_copy(rows_v, out_hbm.at[pl.ds(ch * _CHW, _CHW)])
            return carry

        lax.fori_loop(0, nch, body, 0)

    return k(table, idxm)


def _sc_scatter(exw, wv, idxm, d):
    @functools.partial(
        pl.kernel,
        out_type=(jax.ShapeDtypeStruct((2, N, 16), jnp.float32),
                  jax.ShapeDtypeStruct((2, N, d), jnp.float32)),
        mesh=plsc.VectorSubcoreMesh(**_SC_MESH),
        compiler_params=pltpu.CompilerParams(use_tc_tiling_on_sc=False),
        scratch_types=[
            pltpu.VMEM((_CHW,), jnp.int32),
            pltpu.VMEM((_CHW, 16), jnp.float32),
            pltpu.VMEM((_CHW, d), jnp.float32),
            pltpu.VMEM_SHARED((N, 16), jnp.float32),
            pltpu.VMEM_SHARED((N, d), jnp.float32),
        ],
    )
    def k(ex_hbm, wv_hbm, idxm_hbm, zex_hbm, zwv_hbm, oex_hbm, owv_hbm,
          idx_v, exb, wvb, tex, twv):
        cid = lax.axis_index("c")
        sid = lax.axis_index("s")
        wid = sid * 2 + cid

        @pl.when(sid == 0)
        def _init():
            pltpu.sync_copy(zex_hbm, tex)
            pltpu.sync_copy(zwv_hbm, twv)

        plsc.subcore_barrier()
        nch = (_NCH - wid + (_NW - 1)) // _NW

        def body(t, carry):
            ch = wid + t * _NW
            pltpu.sync_copy(idxm_hbm.at[ch], idx_v)
            pltpu.sync_copy(ex_hbm.at[pl.ds(ch * _CHW, _CHW)], exb)
            pltpu.sync_copy(wv_hbm.at[pl.ds(ch * _CHW, _CHW)], wvb)
            pltpu.sync_copy(exb, tex.at[idx_v], add=True)
            pltpu.sync_copy(wvb, twv.at[idx_v], add=True)
            return carry

        lax.fori_loop(0, nch, body, 0)
        plsc.subcore_barrier()

        @pl.when(sid == 0)
        def _dump():
            pltpu.sync_copy(tex, oex_hbm.at[cid])
            pltpu.sync_copy(twv, owv_hbm.at[cid])

    zex = jnp.zeros((N, 16), jnp.float32)
    zwv = jnp.zeros((N, d), jnp.float32)
    return k(exw, wv, idxm, zex, zwv)


def _ln(v, g, b):
    mu = jnp.mean(v, -1, keepdims=True)
    var = jnp.mean((v - mu) ** 2, -1, keepdims=True)
    return (v - mu) / jnp.sqrt(var + 1e-5) * g + b


def _dot(a, b):
    return jnp.dot(a, b, preferred_element_type=jnp.float32)


def _mlp128_body(h_ref, w1, b1, g1, be1, w2, b2, o_ref):
    v = _dot(h_ref[...], w1[...]) + b1[...]
    v = jnp.maximum(_ln(v, g1[...], be1[...]), 0.0)
    o_ref[...] = _dot(v, w2[...]) + b2[...]


def _mlp128(hin, p):
    return pl.pallas_call(
        _mlp128_body,
        grid=(N // BN,),
        in_specs=[
            pl.BlockSpec((BN, HID), lambda i: (i, 0)),
            pl.BlockSpec((HID, HID), lambda i: (0, 0)),
            pl.BlockSpec((1, HID), lambda i: (0, 0)),
            pl.BlockSpec((1, HID), lambda i: (0, 0)),
            pl.BlockSpec((1, HID), lambda i: (0, 0)),
            pl.BlockSpec((HID, HID), lambda i: (0, 0)),
            pl.BlockSpec((1, HID), lambda i: (0, 0)),
        ],
        out_specs=pl.BlockSpec((BN, HID), lambda i: (i, 0)),
        out_shape=jax.ShapeDtypeStruct((N, HID), jnp.float32),
    )(hin, p['W1'], p['b1'][None, :], p['g'][None, :], p['be'][None, :],
      p['W2'], p['b2'][None, :])


def _rfeat(ea, rel, off, a4, g20):
    d2 = jnp.sum(rel * rel, -1, keepdims=True)
    dist = jnp.sqrt(d2)
    sm = jnp.exp(_COEFF * (dist - off) ** 2)
    return _dot(ea, a4) * _dot(sm, g20)


def _edge1_body(ea_ref, dd_ref, ss_ref,
                off, a4, g20, hs_m, exh,
                w1e, w1r, w1d, w1s, b1,
                gk, bek, w2k, b2k, gv, bev, w2v, b2v,
                eww, ewb,
                ex_ref, wv_ref, rel_ref):
    ea = ea_ref[...]
    hd = dd_ref[:, 0:HID]
    qd = dd_ref[:, HID:2 * HID]
    xd = dd_ref[:, 2 * HID:2 * HID + 16]
    hs = ss_ref[:, 0:HID]
    xs = ss_ref[:, HID:HID + 16]
    rel = xd - xs
    rf = _rfeat(ea, rel, off[...], a4[...], g20[...])
    pre = (_dot(ea, w1e[...]) + _dot(rf, w1r[...])
           + _dot(hd, w1d[...]) + _dot(hs, w1s[...])) + b1[...]
    pk = pre[:, :HID]
    pv = pre[:, HID:]
    k = _dot(jnp.maximum(_ln(pk, gk[...], bek[...]), 0.0), w2k[...]) + b2k[...]
    v = _dot(jnp.maximum(_ln(pv, gv[...], bev[...]), 0.0), w2v[...]) + b2v[...]
    ewl = jnp.sum(rf * eww[...], -1, keepdims=True) + ewb[...]
    v = v * (1.0 / (1.0 + jnp.exp(-ewl)))
    ex = jnp.exp(_dot(qd * k, hs_m[...]) * _ISQ)
    ex_ref[...] = ex
    wv_ref[...] = _dot(ex, exh[...]) * v
    rel_ref[...] = rel


def _edge2_body(ea_ref, rel_ref, dd_ref, ss_ref,
                off, a4, g20, hs_m, p48, q48,
                w1e, w1r, w1d, w1s, b1,
                gk, bek, w2k, b2k, gv, bev, w2v, b2v,
                eww, ewb,
                ex_ref, wv_ref):
    ea = ea_ref[...]
    rel = rel_ref[...]
    hd = dd_ref[:, 0:HID]
    qd = dd_ref[:, HID:2 * HID]
    hs = ss_ref[...]
    rf = _rfeat(ea, rel, off[...], a4[...], g20[...])
    pre = (_dot(ea, w1e[...]) + _dot(rf, w1r[...])
           + _dot(hd, w1d[...]) + _dot(hs, w1s[...])) + b1[...]
    pk = pre[:, :HID]
    pv = pre[:, HID:]
    k = _dot(jnp.maximum(_ln(pk, gk[...], bek[...]), 0.0), w2k[...]) + b2k[...]
    v = _dot(jnp.maximum(_ln(pv, gv[...], bev[...]), 0.0), w2v[...]) + b2v[...]
    ewl = jnp.sum(rf * eww[...], -1, keepdims=True) + ewb[...]
    v = v * (1.0 / (1.0 + jnp.exp(-ewl)))
    ex = jnp.exp(_dot(qd * k, hs_m[...]) * _ISQ)
    ex_ref[...] = ex
    wv_ref[...] = _dot(ex * v, p48[...]) * _dot(rel, q48[...])


def _node1_body(ex0_ref, ex1_ref, wv0_ref, wv1_ref, h_ref, exh,
                w1a, w1b, b1, g1, be1, w2, b2, ho_ref):
    exb = _dot(ex0_ref[...] + ex1_ref[...], exh[...])
    out1 = (wv0_ref[...] + wv1_ref[...]) / (exb + 1e-16)
    pre = _dot(out1, w1a[...]) + _dot(h_ref[...], w1b[...]) + b1[...]
    o = _dot(jnp.maximum(_ln(pre, g1[...], be1[...]), 0.0), w2[...]) + b2[...]
    ho_ref[...] = o + h_ref[...]


def _node2_body(ex0_ref, ex1_ref, wv0_ref, wv1_ref, x_ref, m_ref,
                p48, m48, xo_ref):
    den = _dot(ex0_ref[...] + ex1_ref[...], p48[...])
    o = (wv0_ref[...] + wv1_ref[...]) / (den + 1e-16)
    delta = _dot(o, m48[...])
    xo_ref[...] = x_ref[:, :8] + delta * m_ref[...]


_FULL = lambda r, c: pl.BlockSpec((r, c), lambda i: (0, 0))
_ROWB = lambda b, c: pl.BlockSpec((b, c), lambda i: (i, 0))


def _edge1_call(ea, dd, ss, consts, w):
    return pl.pallas_call(
        _edge1_body,
        grid=(E // BE,),
        in_specs=[
            _ROWB(BE, EFD), _ROWB(BE, 2 * HID + 16), _ROWB(BE, HID + 16),
            _FULL(1, NRG), _FULL(EFD, RFD), _FULL(NRG, RFD),
            _FULL(HID, HEADS), _FULL(HEADS, HID),
            _FULL(EFD, 2 * HID), _FULL(RFD, 2 * HID),
            _FULL(HID, 2 * HID), _FULL(HID, 2 * HID), _FULL(1, 2 * HID),
            _FULL(1, HID), _FULL(1, HID), _FULL(HID, HID), _FULL(1, HID),
            _FULL(1, HID), _FULL(1, HID), _FULL(HID, HID), _FULL(1, HID),
            _FULL(1, RFD), _FULL(1, 1),
        ],
        out_specs=[_ROWB(BE, HEADS), _ROWB(BE, HID), _ROWB(BE, 16)],
        out_shape=[
            jax.ShapeDtypeStruct((E, HEADS), jnp.float32),
            jax.ShapeDtypeStruct((E, HID), jnp.float32),
            jax.ShapeDtypeStruct((E, 16), jnp.float32),
        ],
    )(ea, dd, ss, *consts, *w)


def _edge2_call(ea, rel, dd, ss, consts, w):
    return pl.pallas_call(
        _edge2_body,
        grid=(E // BE,),
        in_specs=[
            _ROWB(BE, EFD), _ROWB(BE, 16), _ROWB(BE, 2 * HID), _ROWB(BE, HID),
            _FULL(1, NRG), _FULL(EFD, RFD), _FULL(NRG, RFD),
            _FULL(HID, HEADS), _FULL(HEADS, 48), _FULL(16, 48),
            _FULL(EFD, 2 * HID), _FULL(RFD, 2 * HID),
            _FULL(HID, 2 * HID), _FULL(HID, 2 * HID), _FULL(1, 2 * HID),
            _FULL(1, HID), _FULL(1, HID), _FULL(HID, HID), _FULL(1, HID),
            _FULL(1, HID), _FULL(1, HID), _FULL(HID, HEADS), _FULL(1, HEADS),
            _FULL(1, RFD), _FULL(1, 1),
        ],
        out_specs=[_ROWB(BE, HEADS), _ROWB(BE, 48)],
        out_shape=[
            jax.ShapeDtypeStruct((E, HEADS), jnp.float32),
            jax.ShapeDtypeStruct((E, 48), jnp.float32),
        ],
    )(ea, rel, dd, ss, *consts, *w)


def _split_w1(p, dout):
    w1 = p['W1']
    return (w1[0:EFD], w1[EFD:EFD + RFD],
            w1[EFD + RFD:EFD + RFD + HID], w1[EFD + RFD + HID:])


def _edge_weights(pk, pv, eww, ewb):
    ke, kr, kd, ks = _split_w1(pk, HID)
    ve, vr, vd, vs = _split_w1(pv, None)
    w1e = jnp.concatenate([ke, ve], 1)
    w1r = jnp.concatenate([kr, vr], 1)
    w1d = jnp.concatenate([kd, vd], 1)
    w1s = jnp.concatenate([ks, vs], 1)
    b1 = jnp.concatenate([pk['b1'], pv['b1']])[None, :]
    return [w1e, w1r, w1d, w1s, b1,
            pk['g'][None, :], pk['be'][None, :], pk['W2'], pk['b2'][None, :],
            pv['g'][None, :], pv['be'][None, :], pv['W2'], pv['b2'][None, :],
            eww.reshape(1, RFD), ewb.reshape(1, 1)]


def kernel(h, x, edge_attr, edge_index, mask_ligand, params):
    p = params
    src = edge_index[0]
    dst = edge_index[1]
    xpad = jnp.concatenate([x, jnp.zeros((N, 13), jnp.float32)], 1)

    off = jnp.asarray(_OFF)[None, :]
    a4 = jnp.asarray(_A4)
    g20 = jnp.asarray(_G20)
    hs_m = jnp.asarray(_HS)
    exh = jnp.asarray(_EXH)
    p48 = jnp.asarray(_P48)
    q48 = jnp.asarray(_Q48)
    m48 = jnp.asarray(_M48)

    srcm = src.reshape(_NCH, _CHW)
    dstm = dst.reshape(_NCH, _CHW)

    q1 = _mlp128(h, p['x2h_hq'])
    dd1 = _sc_gather(jnp.concatenate([h, q1, xpad], 1), dstm, 2 * HID + 16)
    ss1 = _sc_gather(jnp.concatenate([h, xpad], 1), srcm, HID + 16)
    w1 = _edge_weights(p['x2h_hk'], p['x2h_hv'], p['x2h_ew_W'], p['x2h_ew_b'])
    ex1, wv1, rel = _edge1_call(edge_attr, dd1, ss1,
                                (off, a4, g20, hs_m, exh), w1)
    exs1, wvs1 = _sc_scatter(ex1, wv1, dstm, HID)

    po = p['x2h_out']
    hout = pl.pallas_call(
        _node1_body,
        grid=(N // BN,),
        in_specs=[
            _ROWB(BN, HEADS), _ROWB(BN, HEADS), _ROWB(BN, HID), _ROWB(BN, HID),
            _ROWB(BN, HID), _FULL(HEADS, HID),
            _FULL(HID, HID), _FULL(HID, HID), _FULL(1, HID),
            _FULL(1, HID), _FULL(1, HID), _FULL(HID, HID), _FULL(1, HID),
        ],
        out_specs=_ROWB(BN, HID),
        out_shape=jax.ShapeDtypeStruct((N, HID), jnp.float32),
    )(exs1[0], exs1[1], wvs1[0], wvs1[1], h, exh,
      po['W1'][:HID], po['W1'][HID:], po['b1'][None, :],
      po['g'][None, :], po['be'][None, :], po['W2'], po['b2'][None, :])

    q2 = _mlp128(hout, p['h2x_xq'])
    dd2 = _sc_gather(jnp.concatenate([hout, q2], 1), dstm, 2 * HID)
    ss2 = _sc_gather(hout, srcm, HID)
    w2 = _edge_weights(p['h2x_xk'], p['h2x_xv'], p['h2x_ew_W'], p['h2x_ew_b'])
    ex2, wv2 = _edge2_call(edge_attr, rel, dd2, ss2,
                           (off, a4, g20, hs_m, p48, q48), w2)
    exs2, wvs2 = _sc_scatter(ex2, wv2, dstm, 48)

    x8 = pl.pallas_call(
        _node2_body,
        grid=(N // BN,),
        in_specs=[
            _ROWB(BN, HEADS), _ROWB(BN, HEADS), _ROWB(BN, 48), _ROWB(BN, 48),
            _ROWB(BN, 16), _ROWB(BN, 8),
            _FULL(HEADS, 48), _FULL(48, 8),
        ],
        out_specs=_ROWB(BN, 8),
        out_shape=jax.ShapeDtypeStruct((N, 8), jnp.float32),
    )(exs2[0], exs2[1], wvs2[0], wvs2[1], xpad,
      jnp.broadcast_to(mask_ligand[:, None], (N, 8)), p48, m48)

    return (hout, x8[:, :3])
